# Initial kernel scaffold; baseline (speedup 1.0000x reference)
import jax
import jax.numpy as jnp
from jax.experimental import pallas as pl


def kernel(vertices, faces, vertex_features, camera_matrix, view_matrix, face_ids):
    raise NotImplementedError("write your pallas kernel here")


# speedup vs baseline: 8.7477x; 8.7477x over previous
import functools

import jax
import jax.numpy as jnp
from jax import lax
from jax.experimental import pallas as pl
from jax.experimental.pallas import tpu as pltpu
from jax.experimental.pallas import tpu_sc as plsc

IMG_W = 512
IMG_H = 512
P = IMG_W * IMG_H
D = 128

NC = 2
NS = 16
L = 16
NW = NC * NS
PPW = P // NW
B = 128
NCHUNK = PPW // B

VPAD_ROWS = 784
VPAD = VPAD_ROWS * 128


def _proj_body(xc_ref, yc_ref, w_ref, sx_ref, sy_ref):
    xc = xc_ref[...]
    yc = yc_ref[...]
    w = w_ref[...]
    sx_ref[...] = (xc / w + 1.0) * (0.5 * IMG_W)
    sy_ref[...] = (yc / w + 1.0) * (0.5 * IMG_H)


def _project(xc2d, yc2d, w2d):
    return pl.pallas_call(
        _proj_body,
        out_shape=(
            jax.ShapeDtypeStruct((VPAD_ROWS, 128), jnp.float32),
            jax.ShapeDtypeStruct((VPAD_ROWS, 128), jnp.float32),
        ),
    )(xc2d, yc2d, w2d)


_mesh = plsc.VectorSubcoreMesh(
    core_axis_name="c", subcore_axis_name="s", num_cores=NC, num_subcores=NS)


@functools.partial(
    pl.kernel,
    out_type=jax.ShapeDtypeStruct((P, D), jnp.float32),
    mesh=_mesh,
    scratch_types=[
        pltpu.VMEM((B,), jnp.int32),
        pltpu.VMEM((B,), jnp.int32),
        pltpu.VMEM((B,), jnp.int32),
        pltpu.VMEM((B,), jnp.int32),
        pltpu.VMEM((B,), jnp.float32),
        pltpu.VMEM((B,), jnp.float32),
        pltpu.VMEM((B,), jnp.float32),
        pltpu.VMEM((B,), jnp.float32),
        pltpu.VMEM((B,), jnp.float32),
        pltpu.VMEM((B,), jnp.float32),
        pltpu.VMEM((B,), jnp.float32),
        pltpu.VMEM((B,), jnp.float32),
        pltpu.VMEM((B,), jnp.float32),
        pltpu.VMEM((B, D), jnp.float32),
        pltpu.VMEM((B, D), jnp.float32),
        pltpu.VMEM((B, D), jnp.float32),
        pltpu.VMEM((B, D), jnp.float32),
        pltpu.SemaphoreType.DMA,
    ],
)
def _raster(faces0, faces1, faces2, sx_hbm, sy_hbm, fid_hbm, feat_hbm,
            out_hbm, fid_v, i0_v, i1_v, i2_v,
            x0_v, x1_v, x2_v, y0_v, y1_v, y2_v,
            w0_v, w1_v, w2_v, f0_v, f1_v, f2_v, out_v, sem):
    wid = lax.axis_index("s") * NC + lax.axis_index("c")

    def chunk_body(g, _):
        base = wid * PPW + g * B
        pltpu.sync_copy(fid_hbm.at[pl.ds(base, B)], fid_v)
        c0 = pltpu.async_copy(faces0.at[fid_v], i0_v, sem)
        c1 = pltpu.async_copy(faces1.at[fid_v], i1_v, sem)
        c2 = pltpu.async_copy(faces2.at[fid_v], i2_v, sem)
        c0.wait()
        c1.wait()
        c2.wait()
        g0 = pltpu.async_copy(sx_hbm.at[i0_v], x0_v, sem)
        g1 = pltpu.async_copy(sx_hbm.at[i1_v], x1_v, sem)
        g2 = pltpu.async_copy(sx_hbm.at[i2_v], x2_v, sem)
        g3 = pltpu.async_copy(sy_hbm.at[i0_v], y0_v, sem)
        g4 = pltpu.async_copy(sy_hbm.at[i1_v], y1_v, sem)
        g5 = pltpu.async_copy(sy_hbm.at[i2_v], y2_v, sem)
        g0.wait()
        g1.wait()
        g2.wait()
        g3.wait()
        g4.wait()
        g5.wait()
        h0 = pltpu.async_copy(feat_hbm.at[i0_v], f0_v, sem)
        h1 = pltpu.async_copy(feat_hbm.at[i1_v], f1_v, sem)
        h2 = pltpu.async_copy(feat_hbm.at[i2_v], f2_v, sem)

        def wgrp(t, _):
            s = pl.ds(t * L, L)
            pix = base + t * L + lax.iota(jnp.int32, L)
            pxf = (pix & 511).astype(jnp.float32)
            pyf = (pix >> 9).astype(jnp.float32)
            v0x = x0_v[s]
            v0y = y0_v[s]
            e1x = x1_v[s] - v0x
            e1y = y1_v[s] - v0y
            e2x = x2_v[s] - v0x
            e2y = y2_v[s] - v0y
            px = pxf - v0x
            py = pyf - v0y
            dot00 = e2x * e2x + e2y * e2y
            dot01 = e2x * e1x + e2y * e1y
            dot02 = e2x * px + e2y * py
            dot11 = e1x * e1x + e1y * e1y
            dot12 = e1x * px + e1y * py
            inv = 1.0 / (dot00 * dot11 - dot01 * dot01 + 1e-08)
            u = (dot11 * dot02 - dot01 * dot12) * inv
            v = (dot00 * dot12 - dot01 * dot02) * inv
            w0_v[s] = 1.0 - u - v
            w1_v[s] = u
            w2_v[s] = v
            return 0

        lax.fori_loop(0, B // L, wgrp, 0)
        h0.wait()
        h1.wait()
        h2.wait()

        def pixel_grp(t, _):
            wv0 = w0_v[pl.ds(t * L, L)]
            wv1 = w1_v[pl.ds(t * L, L)]
            wv2 = w2_v[pl.ds(t * L, L)]
            for k in range(L):
                j = t * L + k
                a0 = wv0[k]
                a1 = wv1[k]
                a2 = wv2[k]
                for cslice in range(D // L):
                    s = pl.ds(cslice * L, L)
                    out_v[j, s] = (a0 * f0_v[j, s] + a1 * f1_v[j, s]
                                   + a2 * f2_v[j, s])
            return 0

        lax.fori_loop(0, B // L, pixel_grp, 0)
        pltpu.sync_copy(out_v, out_hbm.at[pl.ds(base, B)])
        return 0

    lax.fori_loop(0, NCHUNK, chunk_body, 0)


def kernel(vertices, faces, vertex_features, camera_matrix, view_matrix,
           face_ids):
    V = vertices.shape[0]
    ones = jnp.ones_like(vertices[:, 0:1])
    vertices_homo = jnp.concatenate([vertices, ones], axis=1)
    mvp_matrix = camera_matrix @ view_matrix
    projected = vertices_homo @ mvp_matrix.T
    proj_pad = jnp.zeros((VPAD, 4), jnp.float32).at[:V].set(projected)
    proj_pad = proj_pad.at[V:, 3].set(1.0)
    xc2d = proj_pad[:, 0].reshape(VPAD_ROWS, 128)
    yc2d = proj_pad[:, 1].reshape(VPAD_ROWS, 128)
    w2d = proj_pad[:, 3].reshape(VPAD_ROWS, 128)
    sx2d, sy2d = _project(xc2d, yc2d, w2d)
    sx = sx2d.reshape(VPAD)
    sy = sy2d.reshape(VPAD)
    fi = faces.astype(jnp.int32)
    fid = face_ids.astype(jnp.int32)
    return _raster(fi[:, 0], fi[:, 1], fi[:, 2], sx, sy, fid,
                   vertex_features.astype(jnp.float32))


# speedup vs baseline: 8.9117x; 1.0188x over previous
import functools

import jax
import jax.numpy as jnp
from jax import lax
from jax.experimental import pallas as pl
from jax.experimental.pallas import tpu as pltpu
from jax.experimental.pallas import tpu_sc as plsc

IMG_W = 512
IMG_H = 512
P = IMG_W * IMG_H
D = 128

NC = 2
NS = 16
L = 16
NW = NC * NS
PPW = P // NW
B = 128
NCHUNK = PPW // B

VPAD_ROWS = 784
VPAD = VPAD_ROWS * 128


def _proj_body(xc_ref, yc_ref, w_ref, sx_ref, sy_ref):
    xc = xc_ref[...]
    yc = yc_ref[...]
    w = w_ref[...]
    sx_ref[...] = (xc / w + 1.0) * (0.5 * IMG_W)
    sy_ref[...] = (yc / w + 1.0) * (0.5 * IMG_H)


def _project(xc2d, yc2d, w2d):
    return pl.pallas_call(
        _proj_body,
        out_shape=(
            jax.ShapeDtypeStruct((VPAD_ROWS, 128), jnp.float32),
            jax.ShapeDtypeStruct((VPAD_ROWS, 128), jnp.float32),
        ),
    )(xc2d, yc2d, w2d)


_mesh = plsc.VectorSubcoreMesh(
    core_axis_name="c", subcore_axis_name="s", num_cores=NC, num_subcores=NS)


@functools.partial(
    pl.kernel,
    out_type=jax.ShapeDtypeStruct((P, D), jnp.float32),
    mesh=_mesh,
    scratch_types=[
        pltpu.VMEM((2, B), jnp.int32),
        pltpu.VMEM((2, B), jnp.int32),
        pltpu.VMEM((2, B), jnp.int32),
        pltpu.VMEM((2, B), jnp.int32),
        pltpu.VMEM((2, B), jnp.float32),
        pltpu.VMEM((2, B), jnp.float32),
        pltpu.VMEM((2, B), jnp.float32),
        pltpu.VMEM((2, B), jnp.float32),
        pltpu.VMEM((2, B), jnp.float32),
        pltpu.VMEM((2, B), jnp.float32),
        pltpu.VMEM((2, B), jnp.float32),
        pltpu.VMEM((2, B), jnp.float32),
        pltpu.VMEM((2, B), jnp.float32),
        pltpu.VMEM((2, B, D), jnp.float32),
        pltpu.VMEM((2, B, D), jnp.float32),
        pltpu.VMEM((2, B, D), jnp.float32),
        pltpu.VMEM((B, D), jnp.float32),
        pltpu.SemaphoreType.DMA,
        pltpu.SemaphoreType.DMA,
        pltpu.SemaphoreType.DMA,
    ],
)
def _raster(faces0, faces1, faces2, sx_hbm, sy_hbm, fid_hbm, feat_hbm,
            out_hbm, fid_v, i0_v, i1_v, i2_v,
            x0_v, x1_v, x2_v, y0_v, y1_v, y2_v,
            w0_v, w1_v, w2_v, f0_v, f1_v, f2_v, out_v,
            sem_small, sem_feat, sem_out):
    wid = lax.axis_index("s") * NC + lax.axis_index("c")

    def front_end(g, p):
        base = wid * PPW + g * B
        pltpu.sync_copy(fid_hbm.at[pl.ds(base, B)], fid_v.at[p])
        c0 = pltpu.async_copy(faces0.at[fid_v.at[p]], i0_v.at[p], sem_small)
        c1 = pltpu.async_copy(faces1.at[fid_v.at[p]], i1_v.at[p], sem_small)
        c2 = pltpu.async_copy(faces2.at[fid_v.at[p]], i2_v.at[p], sem_small)
        c0.wait()
        c1.wait()
        c2.wait()
        g0 = pltpu.async_copy(sx_hbm.at[i0_v.at[p]], x0_v.at[p], sem_small)
        g1 = pltpu.async_copy(sx_hbm.at[i1_v.at[p]], x1_v.at[p], sem_small)
        g2 = pltpu.async_copy(sx_hbm.at[i2_v.at[p]], x2_v.at[p], sem_small)
        g3 = pltpu.async_copy(sy_hbm.at[i0_v.at[p]], y0_v.at[p], sem_small)
        g4 = pltpu.async_copy(sy_hbm.at[i1_v.at[p]], y1_v.at[p], sem_small)
        g5 = pltpu.async_copy(sy_hbm.at[i2_v.at[p]], y2_v.at[p], sem_small)
        g0.wait()
        g1.wait()
        g2.wait()
        g3.wait()
        g4.wait()
        g5.wait()

        def wgrp(t, _):
            s = pl.ds(t * L, L)
            pix = base + t * L + lax.iota(jnp.int32, L)
            pxf = (pix & 511).astype(jnp.float32)
            pyf = (pix >> 9).astype(jnp.float32)
            v0x = x0_v[p, s]
            v0y = y0_v[p, s]
            e1x = x1_v[p, s] - v0x
            e1y = y1_v[p, s] - v0y
            e2x = x2_v[p, s] - v0x
            e2y = y2_v[p, s] - v0y
            px = pxf - v0x
            py = pyf - v0y
            dot00 = e2x * e2x + e2y * e2y
            dot01 = e2x * e1x + e2y * e1y
            dot02 = e2x * px + e2y * py
            dot11 = e1x * e1x + e1y * e1y
            dot12 = e1x * px + e1y * py
            inv = 1.0 / (dot00 * dot11 - dot01 * dot01 + 1e-08)
            u = (dot11 * dot02 - dot01 * dot12) * inv
            v = (dot00 * dot12 - dot01 * dot02) * inv
            w0_v[p, s] = 1.0 - u - v
            w1_v[p, s] = u
            w2_v[p, s] = v
            return 0

        lax.fori_loop(0, B // L, wgrp, 0)

    def fire_features(p):
        pltpu.async_copy(feat_hbm.at[i0_v.at[p]], f0_v.at[p], sem_feat)
        pltpu.async_copy(feat_hbm.at[i1_v.at[p]], f1_v.at[p], sem_feat)
        pltpu.async_copy(feat_hbm.at[i2_v.at[p]], f2_v.at[p], sem_feat)

    def wait_features(p):
        pltpu.make_async_copy(feat_hbm.at[i0_v.at[p]], f0_v.at[p],
                              sem_feat).wait()
        pltpu.make_async_copy(feat_hbm.at[i1_v.at[p]], f1_v.at[p],
                              sem_feat).wait()
        pltpu.make_async_copy(feat_hbm.at[i2_v.at[p]], f2_v.at[p],
                              sem_feat).wait()

    front_end(0, 0)
    fire_features(0)

    def chunk_body(g, _):
        p = lax.rem(g, 2)
        q = lax.rem(g + 1, 2)
        base = wid * PPW + g * B

        @pl.when(g + 1 < NCHUNK)
        def _():
            front_end(g + 1, q)

        wait_features(p)

        @pl.when(g + 1 < NCHUNK)
        def _():
            fire_features(q)

        @pl.when(g > 0)
        def _():
            pltpu.make_async_copy(
                out_v, out_hbm.at[pl.ds(base - B, B)], sem_out).wait()

        def pixel_grp(t, _):
            wv0 = w0_v[p, pl.ds(t * L, L)]
            wv1 = w1_v[p, pl.ds(t * L, L)]
            wv2 = w2_v[p, pl.ds(t * L, L)]
            for k in range(L):
                j = t * L + k
                a0 = wv0[k]
                a1 = wv1[k]
                a2 = wv2[k]
                for cslice in range(D // L):
                    s = pl.ds(cslice * L, L)
                    out_v[j, s] = (a0 * f0_v[p, j, s] + a1 * f1_v[p, j, s]
                                   + a2 * f2_v[p, j, s])
            return 0

        lax.fori_loop(0, B // L, pixel_grp, 0)
        pltpu.async_copy(out_v, out_hbm.at[pl.ds(base, B)], sem_out)
        return 0

    lax.fori_loop(0, NCHUNK, chunk_body, 0)
    pltpu.make_async_copy(
        out_v, out_hbm.at[pl.ds(wid * PPW + (NCHUNK - 1) * B, B)],
        sem_out).wait()


def kernel(vertices, faces, vertex_features, camera_matrix, view_matrix,
           face_ids):
    V = vertices.shape[0]
    ones = jnp.ones_like(vertices[:, 0:1])
    vertices_homo = jnp.concatenate([vertices, ones], axis=1)
    mvp_matrix = camera_matrix @ view_matrix
    projected = vertices_homo @ mvp_matrix.T
    proj_pad = jnp.zeros((VPAD, 4), jnp.float32).at[:V].set(projected)
    proj_pad = proj_pad.at[V:, 3].set(1.0)
    xc2d = proj_pad[:, 0].reshape(VPAD_ROWS, 128)
    yc2d = proj_pad[:, 1].reshape(VPAD_ROWS, 128)
    w2d = proj_pad[:, 3].reshape(VPAD_ROWS, 128)
    sx2d, sy2d = _project(xc2d, yc2d, w2d)
    sx = sx2d.reshape(VPAD)
    sy = sy2d.reshape(VPAD)
    fi = faces.astype(jnp.int32)
    fid = face_ids.astype(jnp.int32)
    return _raster(fi[:, 0], fi[:, 1], fi[:, 2], sx, sy, fid,
                   vertex_features.astype(jnp.float32))
